# baseline (device time: 14323 ns/iter reference)
import jax
import jax.numpy as jnp
from jax import lax
from jax.experimental import pallas as pl
from jax.experimental.pallas import tpu as pltpu

N_DEV = 4
B = 2
SQ = 128
D_MODEL = 512
HQ = 4
DH = 64
SKV_LOC = 128
BLK = 64
SCALE = 0.125


def kernel(x, Wq, K_ext, V_ext, Wo):
    Kb = jnp.transpose(K_ext, (0, 2, 3, 1)).astype(jnp.bfloat16)
    Vb = jnp.transpose(V_ext, (0, 2, 3, 1)).astype(jnp.bfloat16)

    def body(x_ref, wq_ref, kb_ref, vb_ref, wo_ref, out_hbm,
             out_v, send_sems, recv_sems, out_sems):
        my = lax.axis_index("i")

        N_CHUNK = B + B * HQ

        def kv_copy(c, t):
            if c < B:
                ref = kb_ref.at[c]
            else:
                b, h = divmod(c - B, HQ)
                ref = vb_ref.at[b, h]
            return pltpu.make_async_remote_copy(
                src_ref=ref,
                dst_ref=ref,
                send_sem=send_sems.at[c * (N_DEV - 1) + max(t - 1, 0)],
                recv_sem=recv_sems.at[c],
                device_id=(t,),
                device_id_type=pl.DeviceIdType.MESH,
            )

        def out_copy(b):
            return pltpu.make_async_copy(
                out_v.at[b], out_hbm.at[b], out_sems.at[b])

        barrier = pltpu.get_barrier_semaphore()

        @pl.when(my != 0)
        def _():
            pl.semaphore_signal(
                barrier, inc=1, device_id=(0,),
                device_id_type=pl.DeviceIdType.MESH,
            )

        @pl.when(my == 0)
        def _():
            pl.semaphore_wait(barrier, N_DEV - 1)
            for c in range(N_CHUNK):
                for t in range(1, N_DEV):
                    kv_copy(c, t).start()

        x2 = x_ref[...].reshape(B * SQ, D_MODEL)
        q_proj = jnp.dot(x2, wq_ref[...],
                         preferred_element_type=jnp.float32)

        row_blk = lax.broadcasted_iota(jnp.int32, (SQ, SKV_LOC), 0) // BLK
        col_blk = lax.broadcasted_iota(jnp.int32, (SQ, SKV_LOC), 1) // BLK
        mask = col_blk <= row_blk

        weights = []
        for b in range(B):
            @pl.when(my != 0)
            def _():
                kv_copy(b, 0).wait_recv()
            for h in range(HQ):
                qh = q_proj[b * SQ:(b + 1) * SQ, h * DH:(h + 1) * DH]
                kh = kb_ref[b, h].astype(jnp.float32)
                s = lax.dot_general(
                    qh, kh, (((1,), (0,)), ((), ())),
                    preferred_element_type=jnp.float32,
                ) * SCALE
                w = jnp.exp(jnp.where(mask, s, -1e9))
                weights.append(w / jnp.sum(w, axis=-1, keepdims=True))

        for b in range(B):
            out_b = jnp.zeros((SQ, D_MODEL), jnp.float32)
            for h in range(HQ):
                @pl.when(my != 0)
                def _():
                    kv_copy(B + b * HQ + h, 0).wait_recv()
                vh = vb_ref[b, h].astype(jnp.float32)
                ctx_h = lax.dot_general(
                    weights[b * HQ + h], vh, (((1,), (1,)), ((), ())),
                    preferred_element_type=jnp.float32,
                )
                out_b = out_b + jnp.dot(
                    ctx_h, wo_ref[h * DH:(h + 1) * DH, :],
                    preferred_element_type=jnp.float32,
                )
            out_v[b] = out_b
            out_copy(b).start()

        for b in range(B):
            out_copy(b).wait()

        @pl.when(my == 0)
        def _():
            for c in range(N_CHUNK):
                for t in range(1, N_DEV):
                    kv_copy(c, t).wait_send()

    out_shape = jax.ShapeDtypeStruct((B, SQ, D_MODEL), jnp.float32)
    return pl.pallas_call(
        body,
        out_shape=out_shape,
        in_specs=[pl.BlockSpec(memory_space=pltpu.VMEM)] * 5,
        out_specs=pl.BlockSpec(memory_space=pl.ANY),
        scratch_shapes=[
            pltpu.VMEM((B, SQ, D_MODEL), jnp.float32),
            pltpu.SemaphoreType.DMA(((B + B * HQ) * (N_DEV - 1),)),
            pltpu.SemaphoreType.DMA((B + B * HQ,)),
            pltpu.SemaphoreType.DMA((B,)),
        ],
        compiler_params=pltpu.CompilerParams(collective_id=0),
    )(x, Wq, Kb, Vb, Wo)


# device time: 14143 ns/iter; 1.0127x vs baseline; 1.0127x over previous
import jax
import jax.numpy as jnp
from jax import lax
from jax.experimental import pallas as pl
from jax.experimental.pallas import tpu as pltpu

N_DEV = 4
B = 2
SQ = 128
D_MODEL = 512
HQ = 4
DH = 64
SKV_LOC = 128
BLK = 64
SCALE = 0.125


def kernel(x, Wq, K_ext, V_ext, Wo):
    Kb = jnp.transpose(K_ext, (0, 2, 3, 1)).astype(jnp.bfloat16)
    Vb = jnp.transpose(V_ext, (0, 2, 3, 1)).astype(jnp.bfloat16)

    def body(x_ref, wq_ref, kb_ref, vb_ref, wo_ref, out_hbm,
             out_v, send_sems, recv_sems, out_sems):
        my = lax.axis_index("i")

        def kv_copy(c, t):
            i, b = divmod(c, B)
            ref = (kb_ref if i == 0 else vb_ref).at[b]
            return pltpu.make_async_remote_copy(
                src_ref=ref,
                dst_ref=ref,
                send_sem=send_sems.at[c * (N_DEV - 1) + max(t - 1, 0)],
                recv_sem=recv_sems.at[c],
                device_id=(t,),
                device_id_type=pl.DeviceIdType.MESH,
            )

        def out_copy(b):
            return pltpu.make_async_copy(
                out_v.at[b], out_hbm.at[b], out_sems.at[b])

        barrier = pltpu.get_barrier_semaphore()

        @pl.when(my != 0)
        def _():
            pl.semaphore_signal(
                barrier, inc=1, device_id=(0,),
                device_id_type=pl.DeviceIdType.MESH,
            )

        @pl.when(my == 0)
        def _():
            pl.semaphore_wait(barrier, N_DEV - 1)
            for c in range(2 * B):
                for t in range(1, N_DEV):
                    kv_copy(c, t).start()

        x2 = x_ref[...].reshape(B * SQ, D_MODEL)
        q_proj = jnp.dot(x2, wq_ref[...],
                         preferred_element_type=jnp.float32)

        row_blk = lax.broadcasted_iota(jnp.int32, (SQ, SKV_LOC), 0) // BLK
        col_blk = lax.broadcasted_iota(jnp.int32, (SQ, SKV_LOC), 1) // BLK
        mask = col_blk <= row_blk

        weights = []
        for b in range(B):
            @pl.when(my != 0)
            def _():
                kv_copy(0 * B + b, 0).wait_recv()
            for h in range(HQ):
                qh = q_proj[b * SQ:(b + 1) * SQ, h * DH:(h + 1) * DH]
                kh = kb_ref[b, h].astype(jnp.float32)
                s = lax.dot_general(
                    qh, kh, (((1,), (0,)), ((), ())),
                    preferred_element_type=jnp.float32,
                ) * SCALE
                w = jnp.exp(jnp.where(mask, s, -1e9))
                weights.append(w / jnp.sum(w, axis=-1, keepdims=True))

        for b in range(B):
            @pl.when(my != 0)
            def _():
                kv_copy(1 * B + b, 0).wait_recv()
            ctx_heads = []
            for h in range(HQ):
                vh = vb_ref[b, h].astype(jnp.float32)
                ctx_heads.append(lax.dot_general(
                    weights[b * HQ + h], vh, (((1,), (1,)), ((), ())),
                    preferred_element_type=jnp.float32,
                ))
            ctx = jnp.concatenate(ctx_heads, axis=1)
            out_v[b] = jnp.dot(ctx, wo_ref[...],
                               preferred_element_type=jnp.float32)
            out_copy(b).start()

        for b in range(B):
            out_copy(b).wait()

        @pl.when(my == 0)
        def _():
            for c in range(2 * B):
                for t in range(1, N_DEV):
                    kv_copy(c, t).wait_send()

    out_shape = jax.ShapeDtypeStruct((B, SQ, D_MODEL), jnp.float32)
    return pl.pallas_call(
        body,
        out_shape=out_shape,
        in_specs=[pl.BlockSpec(memory_space=pltpu.VMEM)] * 5,
        out_specs=pl.BlockSpec(memory_space=pl.ANY),
        scratch_shapes=[
            pltpu.VMEM((B, SQ, D_MODEL), jnp.float32),
            pltpu.SemaphoreType.DMA((2 * B * (N_DEV - 1),)),
            pltpu.SemaphoreType.DMA((2 * B,)),
            pltpu.SemaphoreType.DMA((B,)),
        ],
        compiler_params=pltpu.CompilerParams(collective_id=0),
    )(x, Wq, Kb, Vb, Wo)
